# baseline (speedup 1.0000x reference)
import jax
import jax.numpy as jnp
from jax.experimental import pallas as pl

B, Q, N = 2, 1, 4096
D_LOCAL = 256
K_CLUSTER = 64
K_NEIGH = 50
K_POINTS = 15
KP_EXTENT = 0.12


def _kpconv(points, neigh_idx, feats, kp, W):
    neighbors = points[neigh_idx]
    rel = neighbors - points[:, None, :]
    d = jnp.sqrt(jnp.sum((rel[:, :, None, :] - kp[None, None, :, :]) ** 2, axis=-1) + 1e-12)
    infl = jnp.maximum(0.0, 1.0 - d / KP_EXTENT)
    nf = feats[neigh_idx]
    wf = jnp.einsum('mkp,mkd->mpd', infl, nf)
    return jnp.einsum('mpd,pde->me', wf, W)


def _gate_kernel(o_ref, gw_ref, gb_ref, out_ref):
    o = o_ref[...]
    g = jax.nn.sigmoid(o @ gw_ref[...] + gb_ref[...][None, :])
    out_ref[...] = o * g


def kernel(x, kernel_points, W_simple, res1_u1, res1_kp, res1_u2, res1_sc,
           res2_u1, res2_kp, res2_u2, cluster_w, cluster_b, cluster_w2,
           hidden_w, gating_w, gating_b):
    lrelu = lambda t: jax.nn.leaky_relu(t, 0.1)
    pts = jax.lax.stop_gradient(x.reshape(B, Q * N, 3))

    def knn(p):
        d2 = jnp.sum((p[:, None, :] - p[None, :, :]) ** 2, axis=-1)
        _, idx = jax.lax.top_k(-d2, K_NEIGH)
        return idx

    idx = jax.vmap(knn)(pts)
    idx = idx + (jnp.arange(B) * (Q * N))[:, None, None]
    neigh = idx.reshape(B * Q * N, K_NEIGH)
    points = pts.reshape(B * Q * N, 3)
    feats = jnp.ones((B * Q * N, 1), dtype=jnp.float32)

    f0 = lrelu(_kpconv(points, neigh, feats, kernel_points, W_simple))
    y = lrelu(f0 @ res1_u1)
    y = lrelu(_kpconv(points, neigh, y, kernel_points, res1_kp))
    y = y @ res1_u2
    f1 = lrelu(y + f0 @ res1_sc)
    y = lrelu(f1 @ res2_u1)
    y = lrelu(_kpconv(points, neigh, y, kernel_points, res2_kp))
    y = y @ res2_u2
    f2 = lrelu(y + f1)

    x_frontend = f2.reshape(B, N, D_LOCAL)

    act = jnp.einsum('bnd,dk->bnk', x_frontend, cluster_w) + cluster_b
    act = jax.nn.softmax(act, axis=-1)
    a_sum = jnp.sum(act, axis=1, keepdims=True)
    a = a_sum * cluster_w2
    vlad = jnp.einsum('bnk,bnd->bdk', act, x_frontend) - a
    vlad = vlad / (jnp.linalg.norm(vlad, axis=1, keepdims=True) + 1e-12)
    vlad = vlad.reshape(B, -1)
    vlad = vlad / (jnp.linalg.norm(vlad, axis=1, keepdims=True) + 1e-12)
    out = vlad @ hidden_w
    out = pl.pallas_call(
        _gate_kernel,
        out_shape=jax.ShapeDtypeStruct((B, 256), jnp.float32),
    )(out, gating_w, gating_b)
    return out, x_frontend


# speedup vs baseline: 2.5182x; 2.5182x over previous
import jax
import jax.numpy as jnp
from jax import lax
from jax.experimental import pallas as pl
from jax.experimental.pallas import tpu as pltpu

B, N = 2, 4096
K_NEIGH = 50
KP = 15
KP_EXTENT = 0.12
R = 256
C = 512
NBLK = N // R
NCH = N // C
F32 = jnp.float32


def _lrelu(v):
    return jnp.where(v >= 0, v, 0.1 * v)


def _d2_chunk(pts_ref, ptsT_ref, jc):
    rel = []
    d2 = None
    for c in range(3):
        rows = pts_ref[0, :, c:c + 1]
        cols = ptsT_ref[0, c:c + 1, pl.ds(jc * C, C)]
        diff = rows - cols
        sq = diff * diff
        d2 = sq if d2 is None else d2 + sq
        rel.append(cols - rows)
    return d2, rel


def _knn_body(pts_ref, ptsT_ref, t_ref, js_ref, scr):
    d2 = None
    for c in range(3):
        rows = pts_ref[0, :, c:c + 1]
        cols = ptsT_ref[0, c:c + 1, :]
        diff = rows - cols
        sq = diff * diff
        d2 = sq if d2 is None else d2 + sq
    scr[...] = lax.bitcast_convert_type(d2, jnp.int32)

    lo = jnp.zeros((R, 1), jnp.int32)
    hi = jnp.full((R, 1), 0x40400000, jnp.int32)

    def body(_, carry):
        lo, hi = carry
        mid = lo + lax.shift_right_logical(hi - lo, 1)
        cnt = jnp.sum((scr[...] <= mid).astype(jnp.int32), axis=1,
                      keepdims=True)
        ge = cnt >= K_NEIGH
        return jnp.where(ge, lo, mid + 1), jnp.where(ge, mid, hi)

    lo, hi = lax.fori_loop(0, 31, body, (lo, hi))
    t_ref[0] = lax.bitcast_convert_type(hi, F32)

    a = jnp.sum((scr[...] < hi).astype(jnp.int32), axis=1, keepdims=True)
    kt = K_NEIGH - a

    def jbody(_, carry):
        jlo, jhi = carry
        mid = jlo + lax.shift_right_logical(jhi - jlo, 1)
        iota = lax.broadcasted_iota(jnp.int32, (R, N), 1)
        c = jnp.sum(((scr[...] == hi) & (iota <= mid)).astype(jnp.int32),
                    axis=1, keepdims=True)
        ge = c >= kt
        return jnp.where(ge, jlo, mid + 1), jnp.where(ge, mid, jhi)

    jlo = jnp.zeros((R, 1), jnp.int32)
    jhi = jnp.full((R, 1), N - 1, jnp.int32)
    jlo, jhi = lax.fori_loop(0, 12, jbody, (jlo, jhi))
    js_ref[0] = jhi


def _infl(rel, kp_ref, p, maskf):
    dd = None
    for c in range(3):
        dc = rel[c] - kp_ref[p:p + 1, c:c + 1]
        sq = dc * dc
        dd = sq if dd is None else dd + sq
    d = jnp.sqrt(dd + 1e-12)
    return jnp.maximum(0.0, 1.0 - d / KP_EXTENT) * maskf


def _mask(d2, t_row, js_row, jc):
    jidx = lax.broadcasted_iota(jnp.int32, d2.shape, 1) + jc * C
    keep = (d2 < t_row) | ((d2 == t_row) & (jidx <= js_row))
    return jnp.where(keep, 1.0, 0.0).astype(F32)


def _simple_body(pts_ref, ptsT_ref, t_ref, js_ref, kp_ref, ws_ref, u1_ref,
                 f0_ref, y1_ref):
    t_row = t_ref[0]
    js_row = js_ref[0]
    wf = [jnp.zeros((R, 1), F32) for _ in range(KP)]
    for jc in range(NCH):
        d2, rel = _d2_chunk(pts_ref, ptsT_ref, jc)
        maskf = _mask(d2, t_row, js_row, jc)
        for p in range(KP):
            wf[p] += jnp.sum(_infl(rel, kp_ref, p, maskf), axis=1,
                             keepdims=True)
    wf0 = jnp.concatenate(wf, axis=1)
    f0 = _lrelu(lax.dot_general(wf0, ws_ref[...], (((1,), (0,)), ((), ())),
                                preferred_element_type=F32))
    f0_ref[0] = f0
    y1_ref[0] = _lrelu(lax.dot_general(f0, u1_ref[...],
                                       (((1,), (0,)), ((), ())),
                                       preferred_element_type=F32))


def _make_conv_body(with_sc, with_next):
    def body(pts_ref, ptsT_ref, t_ref, js_ref, kp_ref, yin_ref, fin_ref,
             wkp_ref, u2_ref, *rest):
        if with_sc and with_next:
            sc_ref, un_ref, fout_ref, ynext_ref = rest
        elif with_next:
            un_ref, fout_ref, ynext_ref = rest
        else:
            (fout_ref,) = rest
        t_row = t_ref[0]
        js_row = js_ref[0]
        wf = [jnp.zeros((R, 64), F32) for _ in range(KP)]
        for jc in range(NCH):
            d2, rel = _d2_chunk(pts_ref, ptsT_ref, jc)
            maskf = _mask(d2, t_row, js_row, jc)
            yc = yin_ref[0, pl.ds(jc * C, C), :]
            for p in range(KP):
                s = _infl(rel, kp_ref, p, maskf)
                wf[p] += lax.dot_general(s, yc, (((1,), (0,)), ((), ())),
                                         preferred_element_type=F32)
        wfcat = jnp.concatenate(wf, axis=1)
        y2 = _lrelu(lax.dot_general(wfcat, wkp_ref[...],
                                    (((1,), (0,)), ((), ())),
                                    preferred_element_type=F32))
        y3 = lax.dot_general(y2, u2_ref[...], (((1,), (0,)), ((), ())),
                             preferred_element_type=F32)
        fin = fin_ref[0]
        if with_sc:
            skip = lax.dot_general(fin, sc_ref[...], (((1,), (0,)), ((), ())),
                                   preferred_element_type=F32)
        else:
            skip = fin
        fout = _lrelu(y3 + skip)
        fout_ref[0] = fout
        if with_next:
            ynext_ref[0] = _lrelu(lax.dot_general(fout, un_ref[...],
                                                  (((1,), (0,)), ((), ())),
                                                  preferred_element_type=F32))
    return body


def _vlad_body(x_ref, cw_ref, cb_ref, cw2_ref, vlad_ref):
    xb = x_ref[0]
    act = lax.dot_general(xb, cw_ref[...], (((1,), (0,)), ((), ())),
                          preferred_element_type=F32) + cb_ref[...]
    m = jnp.max(act, axis=1, keepdims=True)
    e = jnp.exp(act - m)
    act = e / jnp.sum(e, axis=1, keepdims=True)
    a_sum = jnp.sum(act, axis=0, keepdims=True)
    vlad = lax.dot_general(xb, act, (((0,), (0,)), ((), ())),
                           preferred_element_type=F32)
    vlad = vlad - a_sum * cw2_ref[...]
    n1 = jnp.sqrt(jnp.sum(vlad * vlad, axis=0, keepdims=True))
    vlad_ref[0] = vlad / (n1 + 1e-12)


def _head_body(v_ref, wh_ref, gw_ref, gb_ref, out_ref):
    v = v_ref[...]
    n2 = jnp.sqrt(jnp.sum(v * v, axis=1, keepdims=True))
    v = v / (n2 + 1e-12)
    o = lax.dot_general(v, wh_ref[...], (((1,), (0,)), ((), ())),
                        preferred_element_type=F32)
    g = lax.dot_general(o, gw_ref[...], (((1,), (0,)), ((), ())),
                        preferred_element_type=F32) + gb_ref[...]
    g = 1.0 / (1.0 + jnp.exp(-g))
    out_ref[...] = o * g


def _full(shape):
    return pl.BlockSpec(shape, lambda b, i: (0,) * len(shape))


def _rows(shape):
    return pl.BlockSpec(shape, lambda b, i: (b, i) + (0,) * (len(shape) - 2))


def _batch(shape):
    return pl.BlockSpec(shape, lambda b, i: (b,) + (0,) * (len(shape) - 1))


def kernel(x, kernel_points, W_simple, res1_u1, res1_kp, res1_u2, res1_sc,
           res2_u1, res2_kp, res2_u2, cluster_w, cluster_b, cluster_w2,
           hidden_w, gating_w, gating_b):
    pts = x.reshape(B, N, 3)
    ptsT = jnp.transpose(pts, (0, 2, 1))
    grid = (B, NBLK)

    t, js = pl.pallas_call(
        _knn_body,
        grid=grid,
        in_specs=[_rows((1, R, 3)), _batch((1, 3, N))],
        out_specs=[_rows((1, R, 1)), _rows((1, R, 1))],
        out_shape=[jax.ShapeDtypeStruct((B, N, 1), F32),
                   jax.ShapeDtypeStruct((B, N, 1), jnp.int32)],
        scratch_shapes=[pltpu.VMEM((R, N), jnp.int32)],
    )(pts, ptsT)

    ws = W_simple.reshape(KP, 64)
    f0, y1 = pl.pallas_call(
        _simple_body,
        grid=grid,
        in_specs=[_rows((1, R, 3)), _batch((1, 3, N)), _rows((1, R, 1)),
                  _rows((1, R, 1)), _full((KP, 3)), _full((KP, 64)),
                  _full((64, 64))],
        out_specs=[_rows((1, R, 64)), _rows((1, R, 64))],
        out_shape=[jax.ShapeDtypeStruct((B, N, 64), F32),
                   jax.ShapeDtypeStruct((B, N, 64), F32)],
    )(pts, ptsT, t, js, kernel_points, ws, res1_u1)

    f1, y2 = pl.pallas_call(
        _make_conv_body(True, True),
        grid=grid,
        in_specs=[_rows((1, R, 3)), _batch((1, 3, N)), _rows((1, R, 1)),
                  _rows((1, R, 1)), _full((KP, 3)), _batch((1, N, 64)),
                  _rows((1, R, 64)), _full((KP * 64, 64)), _full((64, 256)),
                  _full((64, 256)), _full((256, 64))],
        out_specs=[_rows((1, R, 256)), _rows((1, R, 64))],
        out_shape=[jax.ShapeDtypeStruct((B, N, 256), F32),
                   jax.ShapeDtypeStruct((B, N, 64), F32)],
    )(pts, ptsT, t, js, kernel_points, y1, f0, res1_kp.reshape(KP * 64, 64),
      res1_u2, res1_sc, res2_u1)

    f2 = pl.pallas_call(
        _make_conv_body(False, False),
        grid=grid,
        in_specs=[_rows((1, R, 3)), _batch((1, 3, N)), _rows((1, R, 1)),
                  _rows((1, R, 1)), _full((KP, 3)), _batch((1, N, 64)),
                  _rows((1, R, 256)), _full((KP * 64, 64)), _full((64, 256))],
        out_specs=_rows((1, R, 256)),
        out_shape=jax.ShapeDtypeStruct((B, N, 256), F32),
    )(pts, ptsT, t, js, kernel_points, y2, f1, res2_kp.reshape(KP * 64, 64),
      res2_u2)

    vladn = pl.pallas_call(
        _vlad_body,
        grid=(B,),
        in_specs=[pl.BlockSpec((1, N, 256), lambda b: (b, 0, 0)),
                  pl.BlockSpec((256, 64), lambda b: (0, 0)),
                  pl.BlockSpec((1, 64), lambda b: (0, 0)),
                  pl.BlockSpec((256, 64), lambda b: (0, 0))],
        out_specs=pl.BlockSpec((1, 256, 64), lambda b: (b, 0, 0)),
        out_shape=jax.ShapeDtypeStruct((B, 256, 64), F32),
    )(f2, cluster_w, cluster_b.reshape(1, 64), cluster_w2[0])

    flat = vladn.reshape(B, 256 * 64)
    out = pl.pallas_call(
        _head_body,
        out_shape=jax.ShapeDtypeStruct((B, 256), F32),
    )(flat, hidden_w, gating_w, gating_b.reshape(1, 256))

    return out, f2
